# baseline (speedup 1.0000x reference)
import functools

import jax
import jax.numpy as jnp
from jax import lax
from jax.experimental import pallas as pl
from jax.experimental.pallas import tpu as pltpu
from jax.experimental.pallas import tpu_sc as plsc

T_STEPS = 30
BANDS = 5
W = 512
H = 512
ROWS = BANDS * W
NC = 2
NS = 16
L = 16
NW = NC * NS
RPW = ROWS // NW
RG = 1
NG = RPW // RG
VPG = RG * H // L


def _minmax_tc_body(x_ref, o_ref):
    x = x_ref[...]
    mn = jnp.min(x)
    mx = jnp.max(x)
    rows = lax.broadcasted_iota(jnp.int32, (8, 128), 0)
    o_ref[...] = jnp.where(rows == 0, mn, mx)


def _encode_body(in_hbm, mm_hbm, out_hbm,
                 in0, in1, blk0, blk1, tid0, tid1, all_v,
                 is0, is1, os0, os1):
    cid = lax.axis_index("c")
    sid = lax.axis_index("s")
    wid = cid * NS + sid

    pltpu.sync_copy(mm_hbm, all_v)
    mn = all_v[0, pl.ds(0, L)]
    mx_vec = all_v[1, pl.ds(0, L)]
    recip = 1.0 / (mx_vec - mn + 1e-8)
    iota = lax.iota(jnp.int32, L)

    zeros = jnp.zeros((L,), jnp.float32)
    ones = jnp.full((L,), 1.0, jnp.float32)
    ins = (in0, in1)
    blks = (blk0, blk1)
    tids = (tid0, tid1)
    isems = (is0, is1)
    osems = (os0, os1)
    NB = 2

    for blk in blks:
        def zero_t(t, _, blk=blk):
            for r in range(RG):
                for j in range(H // L):
                    blk[t, r, pl.ds(j * L, L)] = zeros
            return 0

        lax.fori_loop(0, T_STEPS, zero_t, 0)

    def in_start(g, b):
        row0 = wid * RPW + g * RG
        pltpu.async_copy(in_hbm.at[pl.ds(row0 * H, RG * H)], ins[b], isems[b])

    def in_wait(b):
        pltpu.make_async_copy(in_hbm.at[pl.ds(0, RG * H)], ins[b],
                              isems[b]).wait()

    def out_start(g, b):
        row0 = wid * RPW + g * RG
        pltpu.async_copy(blks[b], out_hbm.at[:, pl.ds(row0, RG), :], osems[b])

    def out_wait(b):
        pltpu.make_async_copy(blks[b], out_hbm.at[:, pl.ds(0, RG), :],
                              osems[b]).wait()

    def restore(b):
        blk, tid_v = blks[b], tids[b]

        def restore_vec(i, _):
            t = tid_v[pl.ds(i * L, L)]
            r_vec = jnp.full((L,), (i * L) // H, jnp.int32)
            j_vec = ((i * L) % H) + iota
            plsc.store_scatter(blk, [t, r_vec, j_vec], zeros)
            return 0

        lax.fori_loop(0, VPG, restore_vec, 0)

    def encode(b):
        blk, tid_v, in_v = blks[b], tids[b], ins[b]

        def enc_vec(i, _):
            x = in_v[pl.ds(i * L, L)]
            lat = (1.0 - (x - mn) * recip) * (T_STEPS - 1.0)
            t = jnp.clip(lat.astype(jnp.int32), 0, T_STEPS - 1)
            tid_v[pl.ds(i * L, L)] = t
            r_vec = jnp.full((L,), (i * L) // H, jnp.int32)
            j_vec = ((i * L) % H) + iota
            plsc.store_scatter(blk, [t, r_vec, j_vec], ones)
            return 0

        lax.fori_loop(0, VPG, enc_vec, 0)

    for b in range(NB):
        in_start(b, b)

    def pair(p, _):
        for b in range(NB):
            g = p * NB + b
            in_wait(b)

            @pl.when(p >= 1)
            def _():
                out_wait(b)
                restore(b)

            encode(b)
            out_start(g, b)

            @pl.when(p < NG // NB - 1)
            def _():
                in_start(g + NB, b)

        return 0

    lax.fori_loop(0, NG // NB, pair, 0)
    for b in range(NB):
        out_wait(b)


@functools.cache
def _build():
    mesh = plsc.VectorSubcoreMesh(core_axis_name="c", subcore_axis_name="s")
    minmax = pl.pallas_call(
        _minmax_tc_body,
        out_shape=jax.ShapeDtypeStruct((8, 128), jnp.float32),
    )
    encode = pl.kernel(
        _encode_body,
        out_type=jax.ShapeDtypeStruct((T_STEPS, ROWS, H), jnp.float32),
        mesh=mesh,
        compiler_params=pltpu.CompilerParams(needs_layout_passes=False),
        scratch_types=(
            [pltpu.VMEM((RG * H,), jnp.float32)] * 2
            + [pltpu.VMEM((T_STEPS, RG, H), jnp.float32)] * 2
            + [pltpu.VMEM((RG * H,), jnp.int32)] * 2
            + [pltpu.VMEM((8, 128), jnp.float32)]
            + [pltpu.SemaphoreType.DMA] * 4
        ),
    )

    def run(flat2d):
        partials = minmax(flat2d)
        return encode(flat2d.reshape(ROWS * H), partials)

    return run


def kernel(topo_5xwxh):
    out = _build()(topo_5xwxh.reshape(ROWS, H))
    return out.reshape(T_STEPS, BANDS, W, H)


# speedup vs baseline: 1.1636x; 1.1636x over previous
import functools

import jax
import jax.numpy as jnp
from jax import lax
from jax.experimental import pallas as pl
from jax.experimental.pallas import tpu as pltpu
from jax.experimental.pallas import tpu_sc as plsc

T_STEPS = 30
BANDS = 5
W = 512
H = 512
ROWS = BANDS * W
NC = 2
NS = 16
L = 16
NW = NC * NS
RPW = ROWS // NW
RG = 4
NG = RPW // RG
VPG = RG * H // L
VP2 = VPG // 2


def _minmax_tc_body(x_ref, o_ref):
    x = x_ref[...]
    mn = jnp.min(x)
    mx = jnp.max(x)
    rows = lax.broadcasted_iota(jnp.int32, (8, 128), 0)
    o_ref[...] = jnp.where(rows == 0, mn, mx)


def _encode_body(in_hbm, mm_hbm, out_hbm,
                 in0, in1, blk0, blk1, tid0, tid1, all_v,
                 is0, is1, os0, os1):
    cid = lax.axis_index("c")
    sid = lax.axis_index("s")
    wid = cid * NS + sid

    pltpu.sync_copy(mm_hbm.at[pl.ds(0, 2), :], all_v)
    mn = all_v[0, pl.ds(0, L)]
    mx_vec = all_v[1, pl.ds(0, L)]
    recip = 1.0 / (mx_vec - mn + 1e-8)
    iota = lax.iota(jnp.int32, L)

    zeros = jnp.zeros((L,), jnp.float32)
    ones = jnp.full((L,), 1.0, jnp.float32)
    ins = (in0, in1)
    blks = (blk0, blk1)
    tids = (tid0, tid1)
    isems = (is0, is1)
    osems = (os0, os1)
    NB = 2

    for blk in blks:
        def zero_t(t, _, blk=blk):
            for r in range(RG):
                for j in range(H // L):
                    blk[t, r, pl.ds(j * L, L)] = zeros
            return 0

        lax.fori_loop(0, T_STEPS, zero_t, 0)

    def in_start(g, b):
        row0 = wid * RPW + g * RG
        pltpu.async_copy(in_hbm.at[pl.ds(row0 * H, RG * H)], ins[b], isems[b])

    def in_wait(b):
        pltpu.make_async_copy(in_hbm.at[pl.ds(0, RG * H)], ins[b],
                              isems[b]).wait()

    def out_start(g, b):
        row0 = wid * RPW + g * RG
        pltpu.async_copy(blks[b], out_hbm.at[:, pl.ds(row0, RG), :], osems[b])

    def out_wait(b):
        pltpu.make_async_copy(blks[b], out_hbm.at[:, pl.ds(0, RG), :],
                              osems[b]).wait()

    def restore(b):
        blk, tid_v = blks[b], tids[b]

        def restore_pair(i, _):
            o0 = i * 2 * L
            t0, t1 = plsc.unpack(tid_v[pl.ds(o0, 2 * L)],
                                 format=plsc.PackFormat.INTERLEAVED)
            r_vec = jnp.full((L,), o0 // H, jnp.int32)
            j0 = o0 % H
            plsc.store_scatter(blk, [t0, r_vec, j0 + iota], zeros)
            plsc.store_scatter(blk, [t1, r_vec, j0 + L + iota], zeros)
            return 0

        lax.fori_loop(0, VP2, restore_pair, 0)

    def encode(b):
        blk, tid_v, in_v = blks[b], tids[b], ins[b]

        def enc_t(x):
            lat = (1.0 - (x - mn) * recip) * (T_STEPS - 1.0)
            return jnp.clip(lat.astype(jnp.int32), 0, T_STEPS - 1)

        def enc_pair(i, _):
            o0 = i * 2 * L
            t0 = enc_t(in_v[pl.ds(o0, L)])
            t1 = enc_t(in_v[pl.ds(o0 + L, L)])
            tid_v[pl.ds(o0, 2 * L)] = plsc.pack(
                t0, t1, format=plsc.PackFormat.INTERLEAVED)
            r_vec = jnp.full((L,), o0 // H, jnp.int32)
            j0 = o0 % H
            plsc.store_scatter(blk, [t0, r_vec, j0 + iota], ones)
            plsc.store_scatter(blk, [t1, r_vec, j0 + L + iota], ones)
            return 0

        lax.fori_loop(0, VP2, enc_pair, 0)

    for b in range(NB):
        in_start(b, b)

    def pair(p, _):
        for b in range(NB):
            g = p * NB + b
            in_wait(b)

            @pl.when(p >= 1)
            def _():
                out_wait(b)
                restore(b)

            encode(b)
            out_start(g, b)

            @pl.when(p < NG // NB - 1)
            def _():
                in_start(g + NB, b)

        return 0

    lax.fori_loop(0, NG // NB, pair, 0)
    for b in range(NB):
        out_wait(b)


@functools.cache
def _build():
    mesh = plsc.VectorSubcoreMesh(core_axis_name="c", subcore_axis_name="s")
    minmax = pl.pallas_call(
        _minmax_tc_body,
        out_shape=jax.ShapeDtypeStruct((8, 128), jnp.float32),
    )
    encode = pl.kernel(
        _encode_body,
        out_type=jax.ShapeDtypeStruct((T_STEPS, ROWS, H), jnp.float32),
        mesh=mesh,
        compiler_params=pltpu.CompilerParams(needs_layout_passes=False),
        scratch_types=(
            [pltpu.VMEM((RG * H,), jnp.float32)] * 2
            + [pltpu.VMEM((T_STEPS, RG, H), jnp.float32)] * 2
            + [pltpu.VMEM((RG * H,), jnp.int16)] * 2
            + [pltpu.VMEM((2, 128), jnp.float32)]
            + [pltpu.SemaphoreType.DMA] * 4
        ),
    )

    def run(flat2d):
        partials = minmax(flat2d)
        return encode(flat2d.reshape(ROWS * H), partials)

    return run


def kernel(topo_5xwxh):
    out = _build()(topo_5xwxh.reshape(ROWS, H))
    return out.reshape(T_STEPS, BANDS, W, H)
